# baseline (speedup 1.0000x reference)
import functools

import numpy as np

import jax
import jax.numpy as jnp
from jax import lax
from jax.experimental import pallas as pl
from jax.experimental.pallas import tpu as pltpu
from jax.experimental.pallas import tpu_sc as plsc

N_NODES = 50000
N_EDGES = 800000
BATCH = 32

LANES = 16
NW = 32
E_PER_W = N_EDGES // NW
G_FULL = E_PER_W // LANES
TAIL = E_PER_W - G_FULL * LANES
E_BUF = E_PER_W + (LANES - TAIL) % LANES

CHUNK = 7168
N_PAD = 50176

_SIGN = -2147483648
_PAIR_SIGN = -2147450880


def _pack_body(s_ref, packed_ref):
    j = pl.program_id(0)
    sb = s_ref[...]
    col = j * CHUNK + lax.broadcasted_iota(jnp.int32, (BATCH, CHUNK), 1)
    valid = col < N_NODES
    bits = jnp.where(valid & (sb < 0.0), jnp.int32(1), jnp.int32(0))
    row = lax.broadcasted_iota(jnp.int32, (BATCH, CHUNK), 0)
    packed_ref[...] = jnp.sum(bits << row, axis=0)


def _pack(s):
    return pl.pallas_call(
        _pack_body,
        grid=(N_PAD // CHUNK,),
        in_specs=[pl.BlockSpec((BATCH, CHUNK), lambda j: (0, j))],
        out_specs=pl.BlockSpec((CHUNK,), lambda j: (j,)),
        out_shape=jax.ShapeDtypeStruct((N_PAD,), jnp.int32),
    )(s)


def _lin_body(s_ref, lin_ref, lin128_ref):
    j = pl.program_id(0)
    sb = s_ref[...]
    col = j * CHUNK + lax.broadcasted_iota(jnp.int32, (BATCH, CHUNK), 1)
    lb = lin_ref[...].reshape(1, CHUNK)
    prod = jnp.where(col < N_NODES, sb * lb, 0.0)
    part = jnp.sum(prod.reshape(BATCH, CHUNK // 128, 128), axis=1)

    @pl.when(j == 0)
    def _():
        lin128_ref[...] = jnp.zeros_like(lin128_ref)

    lin128_ref[...] += part


def _linear_term(s, linear):
    return pl.pallas_call(
        _lin_body,
        grid=(N_PAD // CHUNK,),
        in_specs=[
            pl.BlockSpec((BATCH, CHUNK), lambda j: (0, j)),
            pl.BlockSpec((CHUNK,), lambda j: (j,)),
        ],
        out_specs=pl.BlockSpec((BATCH, 128), lambda j: (0, 0)),
        out_shape=jax.ShapeDtypeStruct((BATCH, 128), jnp.float32),
    )(s, linear)


@functools.partial(
    pl.kernel,
    out_type=jax.ShapeDtypeStruct((NW, BATCH, LANES), jnp.float32),
    mesh=plsc.VectorSubcoreMesh(core_axis_name="c", subcore_axis_name="s"),
    compiler_params=pltpu.CompilerParams(needs_layout_passes=False),
    scratch_types=[
        pltpu.VMEM((N_PAD,), jnp.int32),
        pltpu.VMEM((E_BUF,), jnp.int32),
        pltpu.VMEM((E_BUF,), jnp.int32),
        pltpu.VMEM((E_BUF,), jnp.float32),
        pltpu.VMEM((BATCH, LANES), jnp.float32),
        pltpu.SemaphoreType.DMA,
        pltpu.SemaphoreType.DMA,
    ],
)
def _quad_energy(packed_hbm, ei_hbm, ej_hbm, q_hbm, out_hbm,
                 table_v, ei_v, ej_v, q_v, row_v, sem_a, sem_b):
    wid = lax.axis_index("s") * 2 + lax.axis_index("c")
    base = wid * E_PER_W

    K = 11
    NBLK = G_FULL // K
    assert NBLK * K == G_FULL
    EPB = K * LANES
    BPC = NBLK // 2
    SPLIT = BPC * EPB
    REST = E_PER_W - SPLIT

    cps_a = [
        pltpu.make_async_copy(packed_hbm, table_v, sem_a),
        pltpu.make_async_copy(ei_hbm.at[pl.ds(base, SPLIT)],
                              ei_v.at[pl.ds(0, SPLIT)], sem_a),
        pltpu.make_async_copy(ej_hbm.at[pl.ds(base, SPLIT)],
                              ej_v.at[pl.ds(0, SPLIT)], sem_a),
        pltpu.make_async_copy(q_hbm.at[pl.ds(base, SPLIT)],
                              q_v.at[pl.ds(0, SPLIT)], sem_a),
    ]
    cps_b = [
        pltpu.make_async_copy(ei_hbm.at[pl.ds(base + SPLIT, REST)],
                              ei_v.at[pl.ds(SPLIT, REST)], sem_b),
        pltpu.make_async_copy(ej_hbm.at[pl.ds(base + SPLIT, REST)],
                              ej_v.at[pl.ds(SPLIT, REST)], sem_b),
        pltpu.make_async_copy(q_hbm.at[pl.ds(base + SPLIT, REST)],
                              q_v.at[pl.ds(SPLIT, REST)], sem_b),
    ]
    for cp in cps_a:
        cp.start()
    for cp in cps_b:
        cp.start()

    zero16 = jnp.zeros((LANES,), jnp.float32)
    for b in range(BATCH):
        row_v[b] = zero16

    for cp in cps_a:
        cp.wait()

    HALF = BATCH // 2

    def block(blk, carry):
        pairs = [jnp.zeros((BATCH,), jnp.bfloat16) for _ in range(HALF)]
        for k in range(K):
            off = (blk * K + k) * LANES
            wi = plsc.load_gather(table_v, [ei_v[pl.ds(off, LANES)]])
            wj = plsc.load_gather(table_v, [ej_v[pl.ds(off, LANES)]])
            x = wi ^ wj
            qv = q_v[pl.ds(off, LANES)]
            qp = plsc.pack(qv, qv, format=plsc.PackFormat.INTERLEAVED)
            q32 = plsc.bitcast(qp, jnp.int32)
            for p in range(HALF):
                sgn = lax.shift_left(x, 15 - p) & _PAIR_SIGN
                pairs[p] = pairs[p] + plsc.bitcast(q32 ^ sgn, jnp.bfloat16)
        for p in range(HALF):
            lo, hi = plsc.unpack(pairs[p], format=plsc.PackFormat.INTERLEAVED)
            plsc.addupdate(row_v.at[p], lo)
            plsc.addupdate(row_v.at[p + HALF], hi)
        return carry

    lax.fori_loop(0, BPC, block, 0)
    for cp in cps_b:
        cp.wait()
    lax.fori_loop(BPC, NBLK, block, 0)

    toff = G_FULL * LANES
    m = lax.iota(jnp.int32, LANES) < TAIL
    wi = plsc.load_gather(table_v, [ei_v[pl.ds(toff, LANES)]], mask=m)
    wj = plsc.load_gather(table_v, [ej_v[pl.ds(toff, LANES)]], mask=m)
    qt = jnp.where(m, q_v[pl.ds(toff, LANES)], 0.0)
    x = wi ^ wj
    qb = plsc.bitcast(qt, jnp.int32)
    for b in range(BATCH):
        contrib = plsc.bitcast(qb ^ (lax.shift_left(x, 31 - b) & _SIGN),
                               jnp.float32)
        plsc.addupdate(row_v.at[b], contrib)

    pltpu.sync_copy(row_v, out_hbm.at[wid])


def kernel(s, linear, quadratic, edge_idx_i, edge_idx_j):
    packed = _pack(s)
    parts = _quad_energy(packed, edge_idx_i.astype(jnp.int32),
                         edge_idx_j.astype(jnp.int32), quadratic)
    lin128 = _linear_term(s, linear)
    return lin128.sum(axis=1) + parts.sum(axis=(0, 2))


# speedup vs baseline: 1.0291x; 1.0291x over previous
import functools

import numpy as np

import jax
import jax.numpy as jnp
from jax import lax
from jax.experimental import pallas as pl
from jax.experimental.pallas import tpu as pltpu
from jax.experimental.pallas import tpu_sc as plsc

N_NODES = 50000
N_EDGES = 800000
BATCH = 32

LANES = 16
NW = 32
E_PER_W = N_EDGES // NW
G_FULL = E_PER_W // LANES
TAIL = E_PER_W - G_FULL * LANES
E_BUF = E_PER_W + (LANES - TAIL) % LANES

CHUNK = 7168
N_PAD = 50176

_SIGN = -2147483648
_PAIR_SIGN = -2147450880


def _pack_body(s_ref, packed_ref):
    sb = s_ref[...]
    bits = jnp.where(sb < 0.0, jnp.int32(1), jnp.int32(0))
    row = lax.broadcasted_iota(jnp.int32, (BATCH, CHUNK), 0)
    packed_ref[...] = jnp.sum(bits << row, axis=0)


def _pack(s):
    return pl.pallas_call(
        _pack_body,
        grid=(N_PAD // CHUNK,),
        in_specs=[pl.BlockSpec((BATCH, CHUNK), lambda j: (0, j))],
        out_specs=pl.BlockSpec((CHUNK,), lambda j: (j,)),
        out_shape=jax.ShapeDtypeStruct((N_PAD,), jnp.int32),
    )(s)


def _lin_body(s_ref, lin_ref, lin128_ref):
    j = pl.program_id(0)
    sb = s_ref[...]
    col = j * CHUNK + lax.broadcasted_iota(jnp.int32, (BATCH, CHUNK), 1)
    lb = lin_ref[...].reshape(1, CHUNK)
    prod = jnp.where(col < N_NODES, sb * lb, 0.0)
    part = jnp.sum(prod.reshape(BATCH, CHUNK // 128, 128), axis=1)

    @pl.when(j == 0)
    def _():
        lin128_ref[...] = jnp.zeros_like(lin128_ref)

    lin128_ref[...] += part


def _linear_term(s, linear):
    return pl.pallas_call(
        _lin_body,
        grid=(N_PAD // CHUNK,),
        in_specs=[
            pl.BlockSpec((BATCH, CHUNK), lambda j: (0, j)),
            pl.BlockSpec((CHUNK,), lambda j: (j,)),
        ],
        out_specs=pl.BlockSpec((BATCH, 128), lambda j: (0, 0)),
        out_shape=jax.ShapeDtypeStruct((BATCH, 128), jnp.float32),
    )(s, linear)


@functools.partial(
    pl.kernel,
    out_type=jax.ShapeDtypeStruct((NW, BATCH, LANES), jnp.float32),
    mesh=plsc.VectorSubcoreMesh(core_axis_name="c", subcore_axis_name="s"),
    compiler_params=pltpu.CompilerParams(needs_layout_passes=False),
    scratch_types=[
        pltpu.VMEM((N_PAD,), jnp.int32),
        pltpu.VMEM((E_BUF,), jnp.int32),
        pltpu.VMEM((E_BUF,), jnp.int32),
        pltpu.VMEM((E_BUF,), jnp.float32),
        pltpu.VMEM((BATCH, LANES), jnp.float32),
        pltpu.SemaphoreType.DMA,
        pltpu.SemaphoreType.DMA,
    ],
)
def _quad_energy(packed_hbm, ei_hbm, ej_hbm, q_hbm, out_hbm,
                 table_v, ei_v, ej_v, q_v, row_v, sem_a, sem_b):
    wid = lax.axis_index("s") * 2 + lax.axis_index("c")
    base = wid * E_PER_W

    K = 22
    NBLK = G_FULL // K
    assert NBLK * K == G_FULL

    cps = [
        pltpu.make_async_copy(packed_hbm, table_v, sem_a),
        pltpu.make_async_copy(ei_hbm.at[pl.ds(base, E_PER_W)],
                              ei_v.at[pl.ds(0, E_PER_W)], sem_a),
        pltpu.make_async_copy(ej_hbm.at[pl.ds(base, E_PER_W)],
                              ej_v.at[pl.ds(0, E_PER_W)], sem_a),
        pltpu.make_async_copy(q_hbm.at[pl.ds(base, E_PER_W)],
                              q_v.at[pl.ds(0, E_PER_W)], sem_b),
    ]
    for cp in cps:
        cp.start()

    zero16 = jnp.zeros((LANES,), jnp.float32)
    for b in range(BATCH):
        row_v[b] = zero16

    for cp in cps:
        cp.wait()

    HALF = BATCH // 2

    def block(blk, carry):
        pairs = [jnp.zeros((BATCH,), jnp.bfloat16) for _ in range(HALF)]
        for k in range(K):
            off = (blk * K + k) * LANES
            wi = plsc.load_gather(table_v, [ei_v[pl.ds(off, LANES)]])
            wj = plsc.load_gather(table_v, [ej_v[pl.ds(off, LANES)]])
            x = wi ^ wj
            qv = q_v[pl.ds(off, LANES)]
            qp = plsc.pack(qv, qv, format=plsc.PackFormat.INTERLEAVED)
            q32 = plsc.bitcast(qp, jnp.int32)
            for p in range(HALF):
                sgn = lax.shift_left(x, 15 - p) & _PAIR_SIGN
                pairs[p] = pairs[p] + plsc.bitcast(q32 ^ sgn, jnp.bfloat16)
        for p in range(HALF):
            lo, hi = plsc.unpack(pairs[p], format=plsc.PackFormat.INTERLEAVED)
            plsc.addupdate(row_v.at[p], lo)
            plsc.addupdate(row_v.at[p + HALF], hi)
        return carry

    lax.fori_loop(0, NBLK, block, 0)

    toff = G_FULL * LANES
    m = lax.iota(jnp.int32, LANES) < TAIL
    wi = plsc.load_gather(table_v, [ei_v[pl.ds(toff, LANES)]], mask=m)
    wj = plsc.load_gather(table_v, [ej_v[pl.ds(toff, LANES)]], mask=m)
    qt = jnp.where(m, q_v[pl.ds(toff, LANES)], 0.0)
    x = wi ^ wj
    qb = plsc.bitcast(qt, jnp.int32)
    for b in range(BATCH):
        contrib = plsc.bitcast(qb ^ (lax.shift_left(x, 31 - b) & _SIGN),
                               jnp.float32)
        plsc.addupdate(row_v.at[b], contrib)

    pltpu.sync_copy(row_v, out_hbm.at[wid])


def kernel(s, linear, quadratic, edge_idx_i, edge_idx_j):
    packed = _pack(s)
    parts = _quad_energy(packed, edge_idx_i.astype(jnp.int32),
                         edge_idx_j.astype(jnp.int32), quadratic)
    lin128 = _linear_term(s, linear)
    return lin128.sum(axis=1) + parts.sum(axis=(0, 2))
